# baseline (speedup 1.0000x reference)
import functools

import jax
import jax.numpy as jnp
from jax import lax
from jax.experimental import pallas as pl
from jax.experimental.pallas import tpu as pltpu
from jax.experimental.pallas import tpu_sc as plsc

K = 1024
D = 256
H = 8
BLK = 256
N_TOK = 4096
TAU = 1.0
COMMIT = 0.25
ENT_RATIO = 0.1
ENT_TEMP = 0.01

SB = 16
NB = K // SB
NW = 32
TPW = N_TOK // NW
CT = 16
L = 16


def _dist_ent_block(x_ref, cb_ref, d_ref, bm_ref, ent_ref, avgp_acc, sacc,
                    *, n_blocks):
    i = pl.program_id(0)

    @pl.when(i == 0)
    def _init():
        avgp_acc[...] = jnp.zeros_like(avgp_acc)
        sacc[0] = 0.0

    x = x_ref[...]
    cb = cb_ref[...]
    ab = lax.dot_general(x, cb, (((1,), (1,)), ((), ())),
                         preferred_element_type=jnp.float32)
    x2 = jnp.sum(x * x, axis=1, keepdims=True)
    b2 = jnp.sum(cb * cb, axis=1)[None, :]
    d = x2 - 2.0 * ab + b2
    d_ref[...] = d
    bm_ref[...] = jnp.min(jnp.reshape(d, (BLK, K // SB, SB)), axis=2)

    a = d * (-1.0 / ENT_TEMP)
    m = jnp.max(a, axis=1, keepdims=True)
    e = jnp.exp(a - m)
    z = jnp.sum(e, axis=1, keepdims=True)
    p = e / z
    s_ent = jnp.log(z[:, 0]) - jnp.sum(e * (a - m), axis=1) / z[:, 0]
    avgp_acc[...] += jnp.sum(p, axis=0, keepdims=True)
    sacc[0] += jnp.sum(s_ent)

    @pl.when(i == n_blocks - 1)
    def _fin():
        navg = 1.0 / N_TOK
        avg_p = avgp_acc[...] * navg
        avg_ent = -jnp.sum(avg_p * jnp.log(avg_p + 1e-5))
        ent_ref[...] = jnp.reshape(
            ENT_RATIO * (sacc[0] * navg - avg_ent), (1, 1))


def _dist_ent(x2d, cb):
    n_blocks = N_TOK // BLK
    kern = functools.partial(_dist_ent_block, n_blocks=n_blocks)
    return pl.pallas_call(
        kern,
        grid=(n_blocks,),
        in_specs=[
            pl.BlockSpec((BLK, D), lambda i: (i, 0)),
            pl.BlockSpec((K, D), lambda i: (0, 0)),
        ],
        out_specs=[
            pl.BlockSpec((BLK, K), lambda i: (i, 0)),
            pl.BlockSpec((BLK, K // SB), lambda i: (i, 0)),
            pl.BlockSpec((1, 1), lambda i: (0, 0)),
        ],
        out_shape=[
            jax.ShapeDtypeStruct((N_TOK, K), jnp.float32),
            jax.ShapeDtypeStruct((N_TOK, K // SB), jnp.float32),
            jax.ShapeDtypeStruct((1, 1), jnp.float32),
        ],
        scratch_shapes=[
            pltpu.VMEM((1, K), jnp.float32),
            pltpu.SMEM((2,), jnp.float32),
        ],
    )(x2d, cb)


def _topk_body(dist_hbm, bm_hbm, ti_hbm, tw_hbm,
               dbuf, bbuf, blkids, cidx, cvals, tibuf, twbuf, sem):
    wid = lax.axis_index("s") * 2 + lax.axis_index("c")
    lane = lax.iota(jnp.int32, L)
    inf_v = jnp.full((L,), jnp.inf, jnp.float32)
    n_chunks = TPW // CT

    def chunk_start(ci, buf):
        tok0 = wid * TPW + ci * CT
        pltpu.async_copy(dist_hbm.at[pl.ds(tok0, CT)], dbuf.at[buf], sem)
        pltpu.async_copy(bm_hbm.at[pl.ds(tok0, CT)], bbuf.at[buf], sem)

    chunk_start(0, 0)

    def chunk_body(ci, _):
        tok0 = wid * TPW + ci * CT
        b = ci % 2

        @pl.when(ci < n_chunks - 1)
        def _prefetch():
            chunk_start(ci + 1, (ci + 1) % 2)

        pltpu.make_async_copy(
            dist_hbm.at[pl.ds(tok0, CT)], dbuf.at[b], sem).wait()
        pltpu.make_async_copy(
            bm_hbm.at[pl.ds(tok0, CT)], bbuf.at[b], sem).wait()

        def token_body(t, _):
            t_v = jnp.broadcast_to(t, (L,))

            bm0 = bbuf[b, t, pl.ds(0, L)]
            bm1 = bbuf[b, t, pl.ds(L, L)]
            bm2 = bbuf[b, t, pl.ds(2 * L, L)]
            bm3 = bbuf[b, t, pl.ds(3 * L, L)]
            s0, _v0 = plsc.sort_key_val(bm0, lane)
            s1, _v1 = plsc.sort_key_val(bm1, lane)
            s2, _v2 = plsc.sort_key_val(bm2, lane)
            s3, _v3 = plsc.sort_key_val(bm3, lane)
            thr = jnp.minimum(jnp.minimum(s0[H - 1], s1[H - 1]),
                              jnp.minimum(s2[H - 1], s3[H - 1]))
            thr_v = jnp.broadcast_to(thr, (L,))

            nb = jnp.int32(0)
            for g, bmg in enumerate((bm0, bm1, bm2, bm3)):
                mg = bmg <= thr_v
                plsc.store_compressed(blkids.at[pl.ds(nb, L)],
                                      lane + g * L, mask=mg)
                nb = nb + plsc.all_reduce_population_count(mg)[0]

            def blk_body(i, cnt):
                blk = plsc.load_gather(blkids, [jnp.broadcast_to(i, (L,))])[0]
                v = dbuf[b, t, pl.ds(blk * SB, L)]
                msk = v <= thr_v
                plsc.store_compressed(cidx.at[pl.ds(cnt, L)],
                                      lane + blk * SB, mask=msk)
                return cnt + plsc.all_reduce_population_count(msk)[0]
            cnt = lax.fori_loop(0, nb, blk_body, jnp.int32(0))

            def fast_path(_):
                iv = cidx[pl.ds(0, L)]
                iv = jnp.where(lane < cnt, iv, jnp.int32(K - 1))
                v = plsc.load_gather(dbuf, [jnp.broadcast_to(b, (L,)),
                                            t_v, iv])
                v = jnp.where(lane < cnt, v, jnp.inf)
                sk2, _ = plsc.sort_key_val(v, lane)
                used = lane >= cnt
                tidx = jnp.zeros((L,), jnp.int32)
                for r in range(H):
                    srv = jnp.broadcast_to(sk2[r], (L,))
                    hit = jnp.logical_and(v == srv, jnp.logical_not(used))
                    p_v = plsc.all_reduce_ffs(hit)
                    used = jnp.logical_or(used, lane == p_v)
                    oi = iv.at[p_v].get(mode="promise_in_bounds")
                    tidx = jnp.where(lane == r, oi, tidx)
                return sk2, tidx

            def gen_path(_):
                plsc.store_scatter(cidx, [lane + cnt],
                                   jnp.full((L,), K - 1, jnp.int32))
                nv = (cnt + 15) // 16

                def fill_body(j, _c):
                    iv = cidx[pl.ds(j * L, L)]
                    v = plsc.load_gather(dbuf, [jnp.broadcast_to(b, (L,)),
                                                t_v, iv])
                    off = jnp.where(lane + j * L < cnt, 0.0, jnp.inf)
                    cvals[pl.ds(j * L, L)] = v + off
                    return 0
                lax.fori_loop(0, nv, fill_body, 0)

                tvals = inf_v
                tidx = jnp.zeros((L,), jnp.int32)
                for r in range(H):
                    def min_body(j, mv):
                        return jnp.minimum(mv, cvals[pl.ds(j * L, L)])
                    mv = lax.fori_loop(0, nv, min_body, inf_v)
                    s_v = jnp.broadcast_to(jnp.min(mv), (L,))

                    def pos_body(j, pv):
                        hit = cvals[pl.ds(j * L, L)] == s_v
                        return jnp.minimum(
                            pv, jnp.where(hit, lane + j * L, jnp.int32(2048)))
                    pv = lax.fori_loop(0, nv, pos_body,
                                       jnp.full((L,), 2048, jnp.int32))
                    p_v = jnp.broadcast_to(jnp.min(pv), (L,))
                    oi = plsc.load_gather(cidx, [p_v])
                    tvals = jnp.where(lane == r, s_v, tvals)
                    tidx = jnp.where(lane == r, oi, tidx)
                    plsc.store_scatter(cvals, [p_v], inf_v, mask=lane == 0)
                return tvals, tidx

            tvals, tidx = lax.cond(cnt <= L, fast_path, gen_path, 0)

            d0 = jnp.broadcast_to(tvals[0], (L,))
            e = jnp.where(lane < H, jnp.exp(d0 - tvals), 0.0)
            tw = e / jnp.broadcast_to(jnp.sum(e), (L,))
            plsc.store_scatter(tibuf, [t_v, lane], tidx, mask=lane < H)
            plsc.store_scatter(twbuf, [t_v, lane], tw, mask=lane < H)
            return 0

        lax.fori_loop(0, CT, token_body, 0)
        pltpu.sync_copy(tibuf, ti_hbm.at[pl.ds(tok0, CT)])
        pltpu.sync_copy(twbuf, tw_hbm.at[pl.ds(tok0, CT)])
        return 0

    lax.fori_loop(0, n_chunks, chunk_body, 0)


def _topk_sc(dist, bmin):
    mesh = plsc.VectorSubcoreMesh(core_axis_name="c", subcore_axis_name="s")
    f = functools.partial(
        pl.kernel,
        mesh=mesh,
        compiler_params=pltpu.CompilerParams(needs_layout_passes=False),
        out_type=[
            jax.ShapeDtypeStruct((N_TOK, H), jnp.int32),
            jax.ShapeDtypeStruct((N_TOK, H), jnp.float32),
        ],
        scratch_types=[
            pltpu.VMEM((2, CT, K), jnp.float32),
            pltpu.VMEM((2, CT, NB), jnp.float32),
            pltpu.VMEM((NB + L,), jnp.int32),
            pltpu.VMEM((K + L,), jnp.int32),
            pltpu.VMEM((K + L,), jnp.float32),
            pltpu.VMEM((CT, H), jnp.int32),
            pltpu.VMEM((CT, H), jnp.float32),
            pltpu.SemaphoreType.DMA,
        ],
    )(_topk_body)
    return f(dist, bmin)


def _finish_block(x_ref, cb_ref, ti_ref, tw_ref, ent_ref,
                  enc_ref, q_ref, loss_ref, sacc, *, n_blocks):
    i = pl.program_id(0)

    @pl.when(i == 0)
    def _init():
        sacc[0] = 0.0

    x = x_ref[...]
    cb = cb_ref[...]
    ti = ti_ref[...]
    tw = tw_ref[...]
    iota_k = lax.broadcasted_iota(jnp.int32, (BLK, K), 1)
    enc = jnp.zeros((BLK, K), jnp.float32)
    for h in range(H):
        enc += jnp.where(iota_k == ti[:, h:h + 1], tw[:, h:h + 1], 0.0)
    enc_ref[...] = enc
    q = lax.dot_general(enc, cb, (((1,), (0,)), ((), ())),
                        preferred_element_type=jnp.float32)
    q_ref[...] = q
    r = q - x
    sacc[0] += jnp.sum(r * r)

    @pl.when(i == n_blocks - 1)
    def _fin():
        mse = sacc[0] * (1.0 / (N_TOK * D))
        loss_ref[...] = jnp.reshape(
            (1.0 + COMMIT) * mse + ent_ref[0, 0], (1, 1))


def _finish(x2d, cb, ti, tw, ent):
    n_blocks = N_TOK // BLK
    kern = functools.partial(_finish_block, n_blocks=n_blocks)
    return pl.pallas_call(
        kern,
        grid=(n_blocks,),
        in_specs=[
            pl.BlockSpec((BLK, D), lambda i: (i, 0)),
            pl.BlockSpec((K, D), lambda i: (0, 0)),
            pl.BlockSpec((BLK, H), lambda i: (i, 0)),
            pl.BlockSpec((BLK, H), lambda i: (i, 0)),
            pl.BlockSpec((1, 1), lambda i: (0, 0)),
        ],
        out_specs=[
            pl.BlockSpec((BLK, K), lambda i: (i, 0)),
            pl.BlockSpec((BLK, D), lambda i: (i, 0)),
            pl.BlockSpec((1, 1), lambda i: (0, 0)),
        ],
        out_shape=[
            jax.ShapeDtypeStruct((N_TOK, K), jnp.float32),
            jax.ShapeDtypeStruct((N_TOK, D), jnp.float32),
            jax.ShapeDtypeStruct((1, 1), jnp.float32),
        ],
        scratch_shapes=[
            pltpu.SMEM((2,), jnp.float32),
        ],
    )(x2d, cb, ti, tw, ent)


@jax.jit
def _vq(x2d, cb):
    dist, bmin, ent = _dist_ent(x2d, cb)
    ti, tw = _topk_sc(dist, bmin)
    enc, q, loss = _finish(x2d, cb, ti, tw, ent)
    return q, loss, ti, tw, enc


def kernel(x, codebook):
    b, t, d = x.shape
    x2d = x.reshape(b * t, d)
    q, loss, ti, tw, enc = _vq(x2d, codebook)
    return (q.reshape(b, t, d), loss[0, 0], ti.reshape(b, t, H),
            tw.reshape(b, t, H), enc.reshape(b, t, K))


# speedup vs baseline: 2.0739x; 2.0739x over previous
import functools

import jax
import jax.numpy as jnp
from jax import lax
from jax.experimental import pallas as pl
from jax.experimental.pallas import tpu as pltpu
from jax.experimental.pallas import tpu_sc as plsc

K = 1024
D = 256
H = 8
BLK = 256
N_TOK = 4096
TAU = 1.0
COMMIT = 0.25
ENT_RATIO = 0.1
ENT_TEMP = 0.01

NB = 128
SE = K // NB
NW = 32
TPW = N_TOK // NW
CT = 16
L = 16


def _dist_ent_block(x_ref, cb_ref, d_ref, bm_ref, ent_ref, avgp_acc, sacc,
                    *, n_blocks):
    i = pl.program_id(0)

    @pl.when(i == 0)
    def _init():
        avgp_acc[...] = jnp.zeros_like(avgp_acc)
        sacc[0] = 0.0

    x = x_ref[...]
    cb = cb_ref[...]
    ab = lax.dot_general(x, cb, (((1,), (1,)), ((), ())),
                         preferred_element_type=jnp.float32)
    x2 = jnp.sum(x * x, axis=1, keepdims=True)
    b2 = jnp.sum(cb * cb, axis=1)[None, :]
    d = x2 - 2.0 * ab + b2
    d_ref[...] = d
    bm = d[:, 0:NB]
    for j in range(1, K // NB):
        bm = jnp.minimum(bm, d[:, NB * j:NB * (j + 1)])
    bm_ref[...] = bm

    a = d * (-1.0 / ENT_TEMP)
    m = jnp.max(a, axis=1, keepdims=True)
    e = jnp.exp(a - m)
    z = jnp.sum(e, axis=1, keepdims=True)
    p = e / z
    s_ent = jnp.log(z[:, 0]) - jnp.sum(e * (a - m), axis=1) / z[:, 0]
    avgp_acc[...] += jnp.sum(p, axis=0, keepdims=True)
    sacc[0] += jnp.sum(s_ent)

    @pl.when(i == n_blocks - 1)
    def _fin():
        navg = 1.0 / N_TOK
        avg_p = avgp_acc[...] * navg
        avg_ent = -jnp.sum(avg_p * jnp.log(avg_p + 1e-5))
        ent_ref[...] = jnp.reshape(
            ENT_RATIO * (sacc[0] * navg - avg_ent), (1, 1))


def _dist_ent(x2d, cb):
    n_blocks = N_TOK // BLK
    kern = functools.partial(_dist_ent_block, n_blocks=n_blocks)
    return pl.pallas_call(
        kern,
        grid=(n_blocks,),
        in_specs=[
            pl.BlockSpec((BLK, D), lambda i: (i, 0)),
            pl.BlockSpec((K, D), lambda i: (0, 0)),
        ],
        out_specs=[
            pl.BlockSpec((BLK, K), lambda i: (i, 0)),
            pl.BlockSpec((BLK, NB), lambda i: (i, 0)),
            pl.BlockSpec((1, 1), lambda i: (0, 0)),
        ],
        out_shape=[
            jax.ShapeDtypeStruct((N_TOK, K), jnp.float32),
            jax.ShapeDtypeStruct((N_TOK, NB), jnp.float32),
            jax.ShapeDtypeStruct((1, 1), jnp.float32),
        ],
        scratch_shapes=[
            pltpu.VMEM((1, K), jnp.float32),
            pltpu.SMEM((2,), jnp.float32),
        ],
    )(x2d, cb)


def _topk_body(dist_hbm, bm_hbm, ti_hbm, tw_hbm,
               dbuf, bbuf, blkids, cidx, cvals, tibuf, twbuf, sem):
    wid = lax.axis_index("s") * 2 + lax.axis_index("c")
    lane = lax.iota(jnp.int32, L)
    lane8 = (lane & 7) * NB
    inf_v = jnp.full((L,), jnp.inf, jnp.float32)
    n_chunks = TPW // CT

    def chunk_start(ci, buf):
        tok0 = wid * TPW + ci * CT
        pltpu.async_copy(dist_hbm.at[pl.ds(tok0, CT)], dbuf.at[buf], sem)
        pltpu.async_copy(bm_hbm.at[pl.ds(tok0, CT)], bbuf.at[buf], sem)

    chunk_start(0, 0)

    def chunk_body(ci, _):
        tok0 = wid * TPW + ci * CT
        b = ci % 2

        @pl.when(ci < n_chunks - 1)
        def _prefetch():
            chunk_start(ci + 1, (ci + 1) % 2)

        pltpu.make_async_copy(
            dist_hbm.at[pl.ds(tok0, CT)], dbuf.at[b], sem).wait()
        pltpu.make_async_copy(
            bm_hbm.at[pl.ds(tok0, CT)], bbuf.at[b], sem).wait()

        def token_body(t, _):
            t_v = jnp.broadcast_to(t, (L,))
            b_v = jnp.broadcast_to(b, (L,))

            bms = [bbuf[b, t, pl.ds(g * L, L)] for g in range(NB // L)]
            p0 = jnp.minimum(jnp.minimum(bms[0], bms[1]),
                             jnp.minimum(bms[2], bms[3]))
            p1 = jnp.minimum(jnp.minimum(bms[4], bms[5]),
                             jnp.minimum(bms[6], bms[7]))
            sk, _sv = plsc.sort_key_val(jnp.minimum(p0, p1), lane)
            thr_v = jnp.broadcast_to(sk[H - 1], (L,))

            nb = jnp.int32(0)
            for g in range(NB // L):
                mg = bms[g] <= thr_v
                plsc.store_compressed(blkids.at[pl.ds(nb, L)],
                                      lane + g * L, mask=mg)
                nb = nb + plsc.all_reduce_population_count(mg)[0]

            def blk_body(i, cnt):
                blk = plsc.load_gather(blkids, [jnp.broadcast_to(i, (L,))])
                kpos = blk + lane8
                v = plsc.load_gather(dbuf, [b_v, t_v, kpos])
                msk = jnp.logical_and(v <= thr_v, lane < SE)
                plsc.store_compressed(cidx.at[pl.ds(cnt, L)], kpos, mask=msk)
                return cnt + plsc.all_reduce_population_count(msk)[0]
            cnt = lax.fori_loop(0, nb, blk_body, jnp.int32(0))

            def fast_path(_):
                iv = cidx[pl.ds(0, L)]
                iv = jnp.where(lane < cnt, iv, jnp.int32(2048))
                siv, _sl = plsc.sort_key_val(iv, lane)
                giv = jnp.minimum(siv, jnp.int32(K - 1))
                v = plsc.load_gather(dbuf, [b_v, t_v, giv])
                v = jnp.where(siv < 2048, v, jnp.inf)
                sk2, _s2 = plsc.sort_key_val(v, lane)
                used = siv >= 2048
                tidx = jnp.zeros((L,), jnp.int32)
                for r in range(H):
                    srv = jnp.broadcast_to(sk2[r], (L,))
                    hit = jnp.logical_and(v == srv, jnp.logical_not(used))
                    p_v = plsc.all_reduce_ffs(hit)
                    used = jnp.logical_or(used, lane == p_v)
                    oi = siv.at[p_v].get(mode="promise_in_bounds")
                    tidx = jnp.where(lane == r, oi, tidx)
                return sk2, tidx

            def gen_path(_):
                plsc.store_scatter(cidx, [lane + cnt],
                                   jnp.full((L,), K - 1, jnp.int32))
                nv = (cnt + 15) // 16

                def fill_body(j, _c):
                    civ = cidx[pl.ds(j * L, L)]
                    v = plsc.load_gather(dbuf, [b_v, t_v, civ])
                    off = jnp.where(lane + j * L < cnt, 0.0, jnp.inf)
                    cvals[pl.ds(j * L, L)] = v + off
                    return 0
                lax.fori_loop(0, nv, fill_body, 0)

                tvals = inf_v
                tidx = jnp.zeros((L,), jnp.int32)
                for r in range(H):
                    def min_body(j, mv):
                        return jnp.minimum(mv, cvals[pl.ds(j * L, L)])
                    mv = lax.fori_loop(0, nv, min_body, inf_v)
                    s_v = jnp.broadcast_to(jnp.min(mv), (L,))

                    def oi_body(j, pv):
                        hit = cvals[pl.ds(j * L, L)] == s_v
                        civ = cidx[pl.ds(j * L, L)]
                        return jnp.minimum(pv,
                                           jnp.where(hit, civ, jnp.int32(K)))
                    pv = lax.fori_loop(0, nv, oi_body,
                                       jnp.full((L,), K, jnp.int32))
                    oi_v = jnp.broadcast_to(jnp.min(pv), (L,))

                    def kill_body(j, _c):
                        hit2 = cidx[pl.ds(j * L, L)] == oi_v
                        plsc.store_scatter(cvals, [lane + j * L], inf_v,
                                           mask=hit2)
                        return 0
                    lax.fori_loop(0, nv, kill_body, 0)
                    tvals = jnp.where(lane == r, s_v, tvals)
                    tidx = jnp.where(lane == r, oi_v, tidx)
                return tvals, tidx

            tvals, tidx = lax.cond(cnt <= L, fast_path, gen_path, 0)

            d0 = jnp.broadcast_to(tvals[0], (L,))
            e = jnp.where(lane < H, jnp.exp(d0 - tvals), 0.0)
            tw = e / jnp.broadcast_to(jnp.sum(e), (L,))
            plsc.store_scatter(tibuf, [t_v, lane], tidx, mask=lane < H)
            plsc.store_scatter(twbuf, [t_v, lane], tw, mask=lane < H)
            return 0

        lax.fori_loop(0, CT, token_body, 0)
        pltpu.sync_copy(tibuf, ti_hbm.at[pl.ds(tok0, CT)])
        pltpu.sync_copy(twbuf, tw_hbm.at[pl.ds(tok0, CT)])
        return 0

    lax.fori_loop(0, n_chunks, chunk_body, 0)


def _topk_sc(dist, bmin):
    mesh = plsc.VectorSubcoreMesh(core_axis_name="c", subcore_axis_name="s")
    f = functools.partial(
        pl.kernel,
        mesh=mesh,
        compiler_params=pltpu.CompilerParams(needs_layout_passes=False),
        out_type=[
            jax.ShapeDtypeStruct((N_TOK, H), jnp.int32),
            jax.ShapeDtypeStruct((N_TOK, H), jnp.float32),
        ],
        scratch_types=[
            pltpu.VMEM((2, CT, K), jnp.float32),
            pltpu.VMEM((2, CT, NB), jnp.float32),
            pltpu.VMEM((NB + L,), jnp.int32),
            pltpu.VMEM((K + L,), jnp.int32),
            pltpu.VMEM((K + L,), jnp.float32),
            pltpu.VMEM((CT, H), jnp.int32),
            pltpu.VMEM((CT, H), jnp.float32),
            pltpu.SemaphoreType.DMA,
        ],
    )(_topk_body)
    return f(dist, bmin)


def _finish_block(x_ref, cb_ref, ti_ref, tw_ref, ent_ref,
                  enc_ref, q_ref, loss_ref, sacc, *, n_blocks):
    i = pl.program_id(0)

    @pl.when(i == 0)
    def _init():
        sacc[0] = 0.0

    x = x_ref[...]
    cb = cb_ref[...]
    ti = ti_ref[...]
    tw = tw_ref[...]
    iota_k = lax.broadcasted_iota(jnp.int32, (BLK, K), 1)
    enc = jnp.zeros((BLK, K), jnp.float32)
    for h in range(H):
        enc += jnp.where(iota_k == ti[:, h:h + 1], tw[:, h:h + 1], 0.0)
    enc_ref[...] = enc
    q = lax.dot_general(enc, cb, (((1,), (0,)), ((), ())),
                        preferred_element_type=jnp.float32)
    q_ref[...] = q
    r = q - x
    sacc[0] += jnp.sum(r * r)

    @pl.when(i == n_blocks - 1)
    def _fin():
        mse = sacc[0] * (1.0 / (N_TOK * D))
        loss_ref[...] = jnp.reshape(
            (1.0 + COMMIT) * mse + ent_ref[0, 0], (1, 1))


def _finish(x2d, cb, ti, tw, ent):
    n_blocks = N_TOK // BLK
    kern = functools.partial(_finish_block, n_blocks=n_blocks)
    return pl.pallas_call(
        kern,
        grid=(n_blocks,),
        in_specs=[
            pl.BlockSpec((BLK, D), lambda i: (i, 0)),
            pl.BlockSpec((K, D), lambda i: (0, 0)),
            pl.BlockSpec((BLK, H), lambda i: (i, 0)),
            pl.BlockSpec((BLK, H), lambda i: (i, 0)),
            pl.BlockSpec((1, 1), lambda i: (0, 0)),
        ],
        out_specs=[
            pl.BlockSpec((BLK, K), lambda i: (i, 0)),
            pl.BlockSpec((BLK, D), lambda i: (i, 0)),
            pl.BlockSpec((1, 1), lambda i: (0, 0)),
        ],
        out_shape=[
            jax.ShapeDtypeStruct((N_TOK, K), jnp.float32),
            jax.ShapeDtypeStruct((N_TOK, D), jnp.float32),
            jax.ShapeDtypeStruct((1, 1), jnp.float32),
        ],
        scratch_shapes=[
            pltpu.SMEM((2,), jnp.float32),
        ],
    )(x2d, cb, ti, tw, ent)


@jax.jit
def _vq(x2d, cb):
    dist, bmin, ent = _dist_ent(x2d, cb)
    ti, tw = _topk_sc(dist, bmin)
    enc, q, loss = _finish(x2d, cb, ti, tw, ent)
    return q, loss, ti, tw, enc


def kernel(x, codebook):
    b, t, d = x.shape
    x2d = x.reshape(b * t, d)
    q, loss, ti, tw, enc = _vq(x2d, codebook)
    return (q.reshape(b, t, d), loss[0, 0], ti.reshape(b, t, H),
            tw.reshape(b, t, H), enc.reshape(b, t, K))


# speedup vs baseline: 2.1329x; 1.0285x over previous
import functools

import jax
import jax.numpy as jnp
from jax import lax
from jax.experimental import pallas as pl
from jax.experimental.pallas import tpu as pltpu
from jax.experimental.pallas import tpu_sc as plsc

K = 1024
D = 256
H = 8
BLK = 256
N_TOK = 4096
TAU = 1.0
COMMIT = 0.25
ENT_RATIO = 0.1
ENT_TEMP = 0.01

NB = 128
SE = K // NB
NW = 32
TPW = N_TOK // NW
CT = 16
L = 16


def _dist_block(x_ref, cb_ref, d_ref, bm_ref):
    x = x_ref[...]
    cb = cb_ref[...]
    ab = lax.dot_general(x, cb, (((1,), (1,)), ((), ())),
                         preferred_element_type=jnp.float32)
    x2 = jnp.sum(x * x, axis=1, keepdims=True)
    b2 = jnp.sum(cb * cb, axis=1)[None, :]
    d = x2 - 2.0 * ab + b2
    d_ref[...] = d
    bm = d[:, 0:NB]
    for j in range(1, K // NB):
        bm = jnp.minimum(bm, d[:, NB * j:NB * (j + 1)])
    bm_ref[...] = bm


def _dist(x2d, cb):
    n_blocks = N_TOK // BLK
    return pl.pallas_call(
        _dist_block,
        grid=(n_blocks,),
        in_specs=[
            pl.BlockSpec((BLK, D), lambda i: (i, 0)),
            pl.BlockSpec((K, D), lambda i: (0, 0)),
        ],
        out_specs=[
            pl.BlockSpec((BLK, K), lambda i: (i, 0)),
            pl.BlockSpec((BLK, NB), lambda i: (i, 0)),
        ],
        out_shape=[
            jax.ShapeDtypeStruct((N_TOK, K), jnp.float32),
            jax.ShapeDtypeStruct((N_TOK, NB), jnp.float32),
        ],
    )(x2d, cb)


def _ent_block(d_ref, ent_ref, avgp_acc, sacc, *, n_blocks):
    i = pl.program_id(0)

    @pl.when(i == 0)
    def _init():
        avgp_acc[...] = jnp.zeros_like(avgp_acc)
        sacc[0] = 0.0

    d = d_ref[...]
    a = d * (-1.0 / ENT_TEMP)
    m = jnp.max(a, axis=1, keepdims=True)
    e = jnp.exp(a - m)
    z = jnp.sum(e, axis=1, keepdims=True)
    p = e / z
    s_ent = jnp.log(z[:, 0]) - jnp.sum(e * (a - m), axis=1) / z[:, 0]
    avgp_acc[...] += jnp.sum(p, axis=0, keepdims=True)
    sacc[0] += jnp.sum(s_ent)

    @pl.when(i == n_blocks - 1)
    def _fin():
        navg = 1.0 / N_TOK
        avg_p = avgp_acc[...] * navg
        avg_ent = -jnp.sum(avg_p * jnp.log(avg_p + 1e-5))
        ent_ref[...] = jnp.reshape(
            ENT_RATIO * (sacc[0] * navg - avg_ent), (1, 1))


def _ent(dist):
    n_blocks = N_TOK // BLK
    kern = functools.partial(_ent_block, n_blocks=n_blocks)
    return pl.pallas_call(
        kern,
        grid=(n_blocks,),
        in_specs=[pl.BlockSpec((BLK, K), lambda i: (i, 0))],
        out_specs=[pl.BlockSpec((1, 1), lambda i: (0, 0))],
        out_shape=[jax.ShapeDtypeStruct((1, 1), jnp.float32)],
        scratch_shapes=[
            pltpu.VMEM((1, K), jnp.float32),
            pltpu.SMEM((2,), jnp.float32),
        ],
    )(dist)[0]


def _topk_body(dist_hbm, bm_hbm, ti_hbm, tw_hbm,
               dbuf, bbuf, blkids, cidx, cvals, tibuf, twbuf, sem):
    wid = lax.axis_index("s") * 2 + lax.axis_index("c")
    lane = lax.iota(jnp.int32, L)
    lane8 = (lane & 7) * NB
    inf_v = jnp.full((L,), jnp.inf, jnp.float32)
    n_chunks = TPW // CT

    def chunk_start(ci, buf):
        tok0 = wid * TPW + ci * CT
        pltpu.async_copy(dist_hbm.at[pl.ds(tok0, CT)], dbuf.at[buf], sem)
        pltpu.async_copy(bm_hbm.at[pl.ds(tok0, CT)], bbuf.at[buf], sem)

    chunk_start(0, 0)

    def chunk_body(ci, _):
        tok0 = wid * TPW + ci * CT
        b = ci % 2

        @pl.when(ci < n_chunks - 1)
        def _prefetch():
            chunk_start(ci + 1, (ci + 1) % 2)

        pltpu.make_async_copy(
            dist_hbm.at[pl.ds(tok0, CT)], dbuf.at[b], sem).wait()
        pltpu.make_async_copy(
            bm_hbm.at[pl.ds(tok0, CT)], bbuf.at[b], sem).wait()

        def token_body(t, _):
            t_v = jnp.broadcast_to(t, (L,))
            b_v = jnp.broadcast_to(b, (L,))

            bms = [bbuf[b, t, pl.ds(g * L, L)] for g in range(NB // L)]
            p0 = jnp.minimum(jnp.minimum(bms[0], bms[1]),
                             jnp.minimum(bms[2], bms[3]))
            p1 = jnp.minimum(jnp.minimum(bms[4], bms[5]),
                             jnp.minimum(bms[6], bms[7]))
            sk, _sv = plsc.sort_key_val(jnp.minimum(p0, p1), lane)
            thr_v = jnp.broadcast_to(sk[H - 1], (L,))

            nb = jnp.int32(0)
            for g in range(NB // L):
                mg = bms[g] <= thr_v
                plsc.store_compressed(blkids.at[pl.ds(nb, L)],
                                      lane + g * L, mask=mg)
                nb = nb + plsc.all_reduce_population_count(mg)[0]

            def blk_body(i, cnt):
                blk = plsc.load_gather(blkids, [jnp.broadcast_to(i, (L,))])
                kpos = blk + lane8
                v = plsc.load_gather(dbuf, [b_v, t_v, kpos])
                msk = jnp.logical_and(v <= thr_v, lane < SE)
                plsc.store_compressed(cidx.at[pl.ds(cnt, L)], kpos, mask=msk)
                return cnt + plsc.all_reduce_population_count(msk)[0]
            cnt = lax.fori_loop(0, nb, blk_body, jnp.int32(0))

            def fast_path(_):
                iv = cidx[pl.ds(0, L)]
                iv = jnp.where(lane < cnt, iv, jnp.int32(2048))
                siv, _sl = plsc.sort_key_val(iv, lane)
                giv = jnp.minimum(siv, jnp.int32(K - 1))
                v = plsc.load_gather(dbuf, [b_v, t_v, giv])
                v = jnp.where(siv < 2048, v, jnp.inf)
                sk2, _s2 = plsc.sort_key_val(v, lane)
                used = siv >= 2048
                tidx = jnp.zeros((L,), jnp.int32)
                for r in range(H):
                    srv = jnp.broadcast_to(sk2[r], (L,))
                    hit = jnp.logical_and(v == srv, jnp.logical_not(used))
                    p_v = plsc.all_reduce_ffs(hit)
                    used = jnp.logical_or(used, lane == p_v)
                    oi = siv.at[p_v].get(mode="promise_in_bounds")
                    tidx = jnp.where(lane == r, oi, tidx)
                return sk2, tidx

            def gen_path(_):
                plsc.store_scatter(cidx, [lane + cnt],
                                   jnp.full((L,), K - 1, jnp.int32))
                nv = (cnt + 15) // 16

                def fill_body(j, _c):
                    civ = cidx[pl.ds(j * L, L)]
                    v = plsc.load_gather(dbuf, [b_v, t_v, civ])
                    off = jnp.where(lane + j * L < cnt, 0.0, jnp.inf)
                    cvals[pl.ds(j * L, L)] = v + off
                    return 0
                lax.fori_loop(0, nv, fill_body, 0)

                tvals = inf_v
                tidx = jnp.zeros((L,), jnp.int32)
                for r in range(H):
                    def min_body(j, mv):
                        return jnp.minimum(mv, cvals[pl.ds(j * L, L)])
                    mv = lax.fori_loop(0, nv, min_body, inf_v)
                    s_v = jnp.broadcast_to(jnp.min(mv), (L,))

                    def oi_body(j, pv):
                        hit = cvals[pl.ds(j * L, L)] == s_v
                        civ = cidx[pl.ds(j * L, L)]
                        return jnp.minimum(pv,
                                           jnp.where(hit, civ, jnp.int32(K)))
                    pv = lax.fori_loop(0, nv, oi_body,
                                       jnp.full((L,), K, jnp.int32))
                    oi_v = jnp.broadcast_to(jnp.min(pv), (L,))

                    def kill_body(j, _c):
                        hit2 = cidx[pl.ds(j * L, L)] == oi_v
                        plsc.store_scatter(cvals, [lane + j * L], inf_v,
                                           mask=hit2)
                        return 0
                    lax.fori_loop(0, nv, kill_body, 0)
                    tvals = jnp.where(lane == r, s_v, tvals)
                    tidx = jnp.where(lane == r, oi_v, tidx)
                return tvals, tidx

            tvals, tidx = lax.cond(cnt <= L, fast_path, gen_path, 0)

            d0 = jnp.broadcast_to(tvals[0], (L,))
            e = jnp.where(lane < H, jnp.exp(d0 - tvals), 0.0)
            tw = e / jnp.broadcast_to(jnp.sum(e), (L,))
            plsc.store_scatter(tibuf, [t_v, lane], tidx, mask=lane < H)
            plsc.store_scatter(twbuf, [t_v, lane], tw, mask=lane < H)
            return 0

        lax.fori_loop(0, CT, token_body, 0)
        pltpu.sync_copy(tibuf, ti_hbm.at[pl.ds(tok0, CT)])
        pltpu.sync_copy(twbuf, tw_hbm.at[pl.ds(tok0, CT)])
        return 0

    lax.fori_loop(0, n_chunks, chunk_body, 0)


def _topk_sc(dist, bmin):
    mesh = plsc.VectorSubcoreMesh(core_axis_name="c", subcore_axis_name="s")
    f = functools.partial(
        pl.kernel,
        mesh=mesh,
        compiler_params=pltpu.CompilerParams(needs_layout_passes=False),
        out_type=[
            jax.ShapeDtypeStruct((N_TOK, H), jnp.int32),
            jax.ShapeDtypeStruct((N_TOK, H), jnp.float32),
        ],
        scratch_types=[
            pltpu.VMEM((2, CT, K), jnp.float32),
            pltpu.VMEM((2, CT, NB), jnp.float32),
            pltpu.VMEM((NB + L,), jnp.int32),
            pltpu.VMEM((K + L,), jnp.int32),
            pltpu.VMEM((K + L,), jnp.float32),
            pltpu.VMEM((CT, H), jnp.int32),
            pltpu.VMEM((CT, H), jnp.float32),
            pltpu.SemaphoreType.DMA,
        ],
    )(_topk_body)
    return f(dist, bmin)


def _finish_block(x_ref, cb_ref, ti_ref, tw_ref, ent_ref,
                  enc_ref, q_ref, loss_ref, sacc, *, n_blocks):
    i = pl.program_id(0)

    @pl.when(i == 0)
    def _init():
        sacc[0] = 0.0

    x = x_ref[...]
    cb = cb_ref[...]
    ti = ti_ref[...]
    tw = tw_ref[...]
    iota_k = lax.broadcasted_iota(jnp.int32, (BLK, K), 1)
    enc = jnp.zeros((BLK, K), jnp.float32)
    for h in range(H):
        enc += jnp.where(iota_k == ti[:, h:h + 1], tw[:, h:h + 1], 0.0)
    enc_ref[...] = enc
    q = lax.dot_general(enc, cb, (((1,), (0,)), ((), ())),
                        preferred_element_type=jnp.float32)
    q_ref[...] = q
    r = q - x
    sacc[0] += jnp.sum(r * r)

    @pl.when(i == n_blocks - 1)
    def _fin():
        mse = sacc[0] * (1.0 / (N_TOK * D))
        loss_ref[...] = jnp.reshape(
            (1.0 + COMMIT) * mse + ent_ref[0, 0], (1, 1))


def _finish(x2d, cb, ti, tw, ent):
    n_blocks = N_TOK // BLK
    kern = functools.partial(_finish_block, n_blocks=n_blocks)
    return pl.pallas_call(
        kern,
        grid=(n_blocks,),
        in_specs=[
            pl.BlockSpec((BLK, D), lambda i: (i, 0)),
            pl.BlockSpec((K, D), lambda i: (0, 0)),
            pl.BlockSpec((BLK, H), lambda i: (i, 0)),
            pl.BlockSpec((BLK, H), lambda i: (i, 0)),
            pl.BlockSpec((1, 1), lambda i: (0, 0)),
        ],
        out_specs=[
            pl.BlockSpec((BLK, K), lambda i: (i, 0)),
            pl.BlockSpec((BLK, D), lambda i: (i, 0)),
            pl.BlockSpec((1, 1), lambda i: (0, 0)),
        ],
        out_shape=[
            jax.ShapeDtypeStruct((N_TOK, K), jnp.float32),
            jax.ShapeDtypeStruct((N_TOK, D), jnp.float32),
            jax.ShapeDtypeStruct((1, 1), jnp.float32),
        ],
        scratch_shapes=[
            pltpu.SMEM((2,), jnp.float32),
        ],
    )(x2d, cb, ti, tw, ent)


@jax.jit
def _vq(x2d, cb):
    dist, bmin = _dist(x2d, cb)
    ti, tw = _topk_sc(dist, bmin)
    ent = _ent(dist)
    enc, q, loss = _finish(x2d, cb, ti, tw, ent)
    return q, loss, ti, tw, enc


def kernel(x, codebook):
    b, t, d = x.shape
    x2d = x.reshape(b * t, d)
    q, loss, ti, tw, enc = _vq(x2d, codebook)
    return (q.reshape(b, t, d), loss[0, 0], ti.reshape(b, t, H),
            tw.reshape(b, t, H), enc.reshape(b, t, K))


# speedup vs baseline: 2.1717x; 1.0182x over previous
import functools

import jax
import jax.numpy as jnp
from jax import lax
from jax.experimental import pallas as pl
from jax.experimental.pallas import tpu as pltpu
from jax.experimental.pallas import tpu_sc as plsc

K = 1024
D = 256
H = 8
BLK = 256
N_TOK = 4096
TAU = 1.0
COMMIT = 0.25
ENT_RATIO = 0.1
ENT_TEMP = 0.01

NB = 128
SE = K // NB
NW = 32
TPW = N_TOK // NW
CT = 32
L = 16


def _dist_block(x_ref, cb_ref, d_ref, bm_ref):
    x = x_ref[...]
    cb = cb_ref[...]
    ab = lax.dot_general(x, cb, (((1,), (1,)), ((), ())),
                         preferred_element_type=jnp.float32)
    x2 = jnp.sum(x * x, axis=1, keepdims=True)
    b2 = jnp.sum(cb * cb, axis=1)[None, :]
    d = x2 - 2.0 * ab + b2
    d_ref[...] = d
    bm = d[:, 0:NB]
    for j in range(1, K // NB):
        bm = jnp.minimum(bm, d[:, NB * j:NB * (j + 1)])
    bm_ref[...] = bm


def _dist(x2d, cb):
    n_blocks = N_TOK // BLK
    return pl.pallas_call(
        _dist_block,
        grid=(n_blocks,),
        in_specs=[
            pl.BlockSpec((BLK, D), lambda i: (i, 0)),
            pl.BlockSpec((K, D), lambda i: (0, 0)),
        ],
        out_specs=[
            pl.BlockSpec((BLK, K), lambda i: (i, 0)),
            pl.BlockSpec((BLK, NB), lambda i: (i, 0)),
        ],
        out_shape=[
            jax.ShapeDtypeStruct((N_TOK, K), jnp.float32),
            jax.ShapeDtypeStruct((N_TOK, NB), jnp.float32),
        ],
    )(x2d, cb)


def _ent_block(d_ref, ent_ref, avgp_acc, sacc, *, n_blocks):
    i = pl.program_id(0)

    @pl.when(i == 0)
    def _init():
        avgp_acc[...] = jnp.zeros_like(avgp_acc)
        sacc[0] = 0.0

    d = d_ref[...]
    a = d * (-1.0 / ENT_TEMP)
    m = jnp.max(a, axis=1, keepdims=True)
    e = jnp.exp(a - m)
    z = jnp.sum(e, axis=1, keepdims=True)
    p = e / z
    s_ent = jnp.log(z[:, 0]) - jnp.sum(e * (a - m), axis=1) / z[:, 0]
    avgp_acc[...] += jnp.sum(p, axis=0, keepdims=True)
    sacc[0] += jnp.sum(s_ent)

    @pl.when(i == n_blocks - 1)
    def _fin():
        navg = 1.0 / N_TOK
        avg_p = avgp_acc[...] * navg
        avg_ent = -jnp.sum(avg_p * jnp.log(avg_p + 1e-5))
        ent_ref[...] = jnp.reshape(
            ENT_RATIO * (sacc[0] * navg - avg_ent), (1, 1))


def _ent(dist):
    n_blocks = N_TOK // BLK
    kern = functools.partial(_ent_block, n_blocks=n_blocks)
    return pl.pallas_call(
        kern,
        grid=(n_blocks,),
        in_specs=[pl.BlockSpec((BLK, K), lambda i: (i, 0))],
        out_specs=[pl.BlockSpec((1, 1), lambda i: (0, 0))],
        out_shape=[jax.ShapeDtypeStruct((1, 1), jnp.float32)],
        scratch_shapes=[
            pltpu.VMEM((1, K), jnp.float32),
            pltpu.SMEM((2,), jnp.float32),
        ],
    )(dist)[0]


def _topk_body(dist_hbm, bm_hbm, ti_hbm, tw_hbm,
               dbuf, bbuf, blkids, cidx, cvals, tibuf, twbuf, sem):
    wid = lax.axis_index("s") * 2 + lax.axis_index("c")
    lane = lax.iota(jnp.int32, L)
    lane8 = (lane & 7) * NB
    inf_v = jnp.full((L,), jnp.inf, jnp.float32)
    n_chunks = TPW // CT

    def chunk_start(ci, buf):
        tok0 = wid * TPW + ci * CT
        pltpu.async_copy(dist_hbm.at[pl.ds(tok0, CT)], dbuf.at[buf], sem)
        pltpu.async_copy(bm_hbm.at[pl.ds(tok0, CT)], bbuf.at[buf], sem)

    chunk_start(0, 0)

    def chunk_body(ci, _):
        tok0 = wid * TPW + ci * CT
        b = ci % 2

        @pl.when(ci < n_chunks - 1)
        def _prefetch():
            chunk_start(ci + 1, (ci + 1) % 2)

        pltpu.make_async_copy(
            dist_hbm.at[pl.ds(tok0, CT)], dbuf.at[b], sem).wait()
        pltpu.make_async_copy(
            bm_hbm.at[pl.ds(tok0, CT)], bbuf.at[b], sem).wait()

        def token_body(t, _):
            t_v = jnp.broadcast_to(t, (L,))
            b_v = jnp.broadcast_to(b, (L,))

            bms = [bbuf[b, t, pl.ds(g * L, L)] for g in range(NB // L)]
            p0 = jnp.minimum(jnp.minimum(bms[0], bms[1]),
                             jnp.minimum(bms[2], bms[3]))
            p1 = jnp.minimum(jnp.minimum(bms[4], bms[5]),
                             jnp.minimum(bms[6], bms[7]))
            sk, _sv = plsc.sort_key_val(jnp.minimum(p0, p1), lane)
            thr_v = jnp.broadcast_to(sk[H - 1], (L,))

            nb = jnp.int32(0)
            for g in range(NB // L):
                mg = bms[g] <= thr_v
                plsc.store_compressed(blkids.at[pl.ds(nb, L)],
                                      lane + g * L, mask=mg)
                nb = nb + plsc.all_reduce_population_count(mg)[0]

            def blk_body(i, cnt):
                blk = plsc.load_gather(blkids, [jnp.broadcast_to(i, (L,))])
                kpos = blk + lane8
                v = plsc.load_gather(dbuf, [b_v, t_v, kpos])
                msk = jnp.logical_and(v <= thr_v, lane < SE)
                plsc.store_compressed(cidx.at[pl.ds(cnt, L)], kpos, mask=msk)
                return cnt + plsc.all_reduce_population_count(msk)[0]
            cnt = lax.fori_loop(0, nb, blk_body, jnp.int32(0))

            def fast_path(_):
                iv = cidx[pl.ds(0, L)]
                giv = jnp.where(lane < cnt, iv, 0)
                v = plsc.load_gather(dbuf, [b_v, t_v, giv])
                v = jnp.where(lane < cnt, v, jnp.inf)
                iv2 = jnp.where(lane < cnt, iv, jnp.int32(2048))
                sk2, sidx = plsc.sort_key_val(v, iv2)
                nxt = sk2.at[jnp.minimum(lane + 1, jnp.int32(L - 1))].get(
                    mode="promise_in_bounds")
                tiem = jnp.logical_and(sk2 == nxt, lane < H)
                anytie = plsc.all_reduce_population_count(tiem)[0]

                def notie(_a):
                    return sk2, sidx

                def tiecase(_a):
                    siv, _sl = plsc.sort_key_val(iv2, lane)
                    gv = jnp.minimum(siv, jnp.int32(K - 1))
                    vv = plsc.load_gather(dbuf, [b_v, t_v, gv])
                    vv = jnp.where(siv < 2048, vv, jnp.inf)
                    vs, _s2 = plsc.sort_key_val(vv, lane)
                    used = siv >= 2048
                    tidx = jnp.zeros((L,), jnp.int32)
                    for r in range(H):
                        srv = jnp.broadcast_to(vs[r], (L,))
                        hit = jnp.logical_and(vv == srv,
                                              jnp.logical_not(used))
                        p_v = plsc.all_reduce_ffs(hit)
                        used = jnp.logical_or(used, lane == p_v)
                        oi = siv.at[p_v].get(mode="promise_in_bounds")
                        tidx = jnp.where(lane == r, oi, tidx)
                    return vs, tidx

                return lax.cond(anytie == 0, notie, tiecase, 0)

            def gen_path(_):
                plsc.store_scatter(cidx, [lane + cnt],
                                   jnp.full((L,), K - 1, jnp.int32))
                nv = (cnt + 15) // 16

                def fill_body(j, _c):
                    civ = cidx[pl.ds(j * L, L)]
                    v = plsc.load_gather(dbuf, [b_v, t_v, civ])
                    off = jnp.where(lane + j * L < cnt, 0.0, jnp.inf)
                    cvals[pl.ds(j * L, L)] = v + off
                    return 0
                lax.fori_loop(0, nv, fill_body, 0)

                tvals = inf_v
                tidx = jnp.zeros((L,), jnp.int32)
                for r in range(H):
                    def min_body(j, mv):
                        return jnp.minimum(mv, cvals[pl.ds(j * L, L)])
                    mv = lax.fori_loop(0, nv, min_body, inf_v)
                    s_v = jnp.broadcast_to(jnp.min(mv), (L,))

                    def oi_body(j, pv):
                        hit = cvals[pl.ds(j * L, L)] == s_v
                        civ = cidx[pl.ds(j * L, L)]
                        return jnp.minimum(pv,
                                           jnp.where(hit, civ, jnp.int32(K)))
                    pv = lax.fori_loop(0, nv, oi_body,
                                       jnp.full((L,), K, jnp.int32))
                    oi_v = jnp.broadcast_to(jnp.min(pv), (L,))

                    def kill_body(j, _c):
                        hit2 = cidx[pl.ds(j * L, L)] == oi_v
                        plsc.store_scatter(cvals, [lane + j * L], inf_v,
                                           mask=hit2)
                        return 0
                    lax.fori_loop(0, nv, kill_body, 0)
                    tvals = jnp.where(lane == r, s_v, tvals)
                    tidx = jnp.where(lane == r, oi_v, tidx)
                return tvals, tidx

            tvals, tidx = lax.cond(cnt <= L, fast_path, gen_path, 0)

            d0 = jnp.broadcast_to(tvals[0], (L,))
            e = jnp.where(lane < H, jnp.exp(d0 - tvals), 0.0)
            tw = e / jnp.broadcast_to(jnp.sum(e), (L,))
            plsc.store_scatter(tibuf, [t_v, lane], tidx, mask=lane < H)
            plsc.store_scatter(twbuf, [t_v, lane], tw, mask=lane < H)
            return 0

        lax.fori_loop(0, CT, token_body, 0)
        pltpu.sync_copy(tibuf, ti_hbm.at[pl.ds(tok0, CT)])
        pltpu.sync_copy(twbuf, tw_hbm.at[pl.ds(tok0, CT)])
        return 0

    lax.fori_loop(0, n_chunks, chunk_body, 0)


def _topk_sc(dist, bmin):
    mesh = plsc.VectorSubcoreMesh(core_axis_name="c", subcore_axis_name="s")
    f = functools.partial(
        pl.kernel,
        mesh=mesh,
        compiler_params=pltpu.CompilerParams(needs_layout_passes=False),
        out_type=[
            jax.ShapeDtypeStruct((N_TOK, H), jnp.int32),
            jax.ShapeDtypeStruct((N_TOK, H), jnp.float32),
        ],
        scratch_types=[
            pltpu.VMEM((2, CT, K), jnp.float32),
            pltpu.VMEM((2, CT, NB), jnp.float32),
            pltpu.VMEM((NB + L,), jnp.int32),
            pltpu.VMEM((K + L,), jnp.int32),
            pltpu.VMEM((K + L,), jnp.float32),
            pltpu.VMEM((CT, H), jnp.int32),
            pltpu.VMEM((CT, H), jnp.float32),
            pltpu.SemaphoreType.DMA,
        ],
    )(_topk_body)
    return f(dist, bmin)


def _finish_block(x_ref, cb_ref, ti_ref, tw_ref, ent_ref,
                  enc_ref, q_ref, loss_ref, sacc, *, n_blocks):
    i = pl.program_id(0)

    @pl.when(i == 0)
    def _init():
        sacc[0] = 0.0

    x = x_ref[...]
    cb = cb_ref[...]
    ti = ti_ref[...]
    tw = tw_ref[...]
    iota_k = lax.broadcasted_iota(jnp.int32, (BLK, K), 1)
    enc = jnp.zeros((BLK, K), jnp.float32)
    for h in range(H):
        enc += jnp.where(iota_k == ti[:, h:h + 1], tw[:, h:h + 1], 0.0)
    enc_ref[...] = enc
    q = lax.dot_general(enc, cb, (((1,), (0,)), ((), ())),
                        preferred_element_type=jnp.float32)
    q_ref[...] = q
    r = q - x
    sacc[0] += jnp.sum(r * r)

    @pl.when(i == n_blocks - 1)
    def _fin():
        mse = sacc[0] * (1.0 / (N_TOK * D))
        loss_ref[...] = jnp.reshape(
            (1.0 + COMMIT) * mse + ent_ref[0, 0], (1, 1))


def _finish(x2d, cb, ti, tw, ent):
    n_blocks = N_TOK // BLK
    kern = functools.partial(_finish_block, n_blocks=n_blocks)
    return pl.pallas_call(
        kern,
        grid=(n_blocks,),
        in_specs=[
            pl.BlockSpec((BLK, D), lambda i: (i, 0)),
            pl.BlockSpec((K, D), lambda i: (0, 0)),
            pl.BlockSpec((BLK, H), lambda i: (i, 0)),
            pl.BlockSpec((BLK, H), lambda i: (i, 0)),
            pl.BlockSpec((1, 1), lambda i: (0, 0)),
        ],
        out_specs=[
            pl.BlockSpec((BLK, K), lambda i: (i, 0)),
            pl.BlockSpec((BLK, D), lambda i: (i, 0)),
            pl.BlockSpec((1, 1), lambda i: (0, 0)),
        ],
        out_shape=[
            jax.ShapeDtypeStruct((N_TOK, K), jnp.float32),
            jax.ShapeDtypeStruct((N_TOK, D), jnp.float32),
            jax.ShapeDtypeStruct((1, 1), jnp.float32),
        ],
        scratch_shapes=[
            pltpu.SMEM((2,), jnp.float32),
        ],
    )(x2d, cb, ti, tw, ent)


@jax.jit
def _vq(x2d, cb):
    dist, bmin = _dist(x2d, cb)
    ti, tw = _topk_sc(dist, bmin)
    ent = _ent(dist)
    enc, q, loss = _finish(x2d, cb, ti, tw, ent)
    return q, loss, ti, tw, enc


def kernel(x, codebook):
    b, t, d = x.shape
    x2d = x.reshape(b * t, d)
    q, loss, ti, tw, enc = _vq(x2d, codebook)
    return (q.reshape(b, t, d), loss[0, 0], ti.reshape(b, t, H),
            tw.reshape(b, t, H), enc.reshape(b, t, K))


# speedup vs baseline: 2.2502x; 1.0362x over previous
import functools

import jax
import jax.numpy as jnp
from jax import lax
from jax.experimental import pallas as pl
from jax.experimental.pallas import tpu as pltpu
from jax.experimental.pallas import tpu_sc as plsc

K = 1024
D = 256
H = 8
BLK = 256
N_TOK = 4096
TAU = 1.0
COMMIT = 0.25
ENT_RATIO = 0.1
ENT_TEMP = 0.01

NB = 128
SE = K // NB
NW = 32
TPW = N_TOK // NW
CT = 32
L = 16


def _dist_block(x_ref, cb_ref, d_ref, bm_ref):
    x = x_ref[...]
    cb = cb_ref[...]
    ab = lax.dot_general(x, cb, (((1,), (1,)), ((), ())),
                         preferred_element_type=jnp.float32)
    x2 = jnp.sum(x * x, axis=1, keepdims=True)
    b2 = jnp.sum(cb * cb, axis=1)[None, :]
    d = x2 - 2.0 * ab + b2
    d_ref[...] = d
    bm = d[:, 0:NB]
    for j in range(1, K // NB):
        bm = jnp.minimum(bm, d[:, NB * j:NB * (j + 1)])
    bm_ref[...] = bm


def _dist(x2d, cb):
    n_blocks = N_TOK // BLK
    return pl.pallas_call(
        _dist_block,
        grid=(n_blocks,),
        in_specs=[
            pl.BlockSpec((BLK, D), lambda i: (i, 0)),
            pl.BlockSpec((K, D), lambda i: (0, 0)),
        ],
        out_specs=[
            pl.BlockSpec((BLK, K), lambda i: (i, 0)),
            pl.BlockSpec((BLK, NB), lambda i: (i, 0)),
        ],
        out_shape=[
            jax.ShapeDtypeStruct((N_TOK, K), jnp.float32),
            jax.ShapeDtypeStruct((N_TOK, NB), jnp.float32),
        ],
    )(x2d, cb)


def _ent_block(d_ref, ent_ref, avgp_acc, sacc, *, n_blocks):
    i = pl.program_id(0)

    @pl.when(i == 0)
    def _init():
        avgp_acc[...] = jnp.zeros_like(avgp_acc)
        sacc[0] = 0.0

    d = d_ref[...]
    a = d * (-1.0 / ENT_TEMP)
    m = jnp.max(a, axis=1, keepdims=True)
    e = jnp.exp(a - m)
    z = jnp.sum(e, axis=1, keepdims=True)
    p = e / z
    s_ent = jnp.log(z[:, 0]) - jnp.sum(e * (a - m), axis=1) / z[:, 0]
    avgp_acc[...] += jnp.sum(p, axis=0, keepdims=True)
    sacc[0] += jnp.sum(s_ent)

    @pl.when(i == n_blocks - 1)
    def _fin():
        navg = 1.0 / N_TOK
        avg_p = avgp_acc[...] * navg
        avg_ent = -jnp.sum(avg_p * jnp.log(avg_p + 1e-5))
        ent_ref[...] = jnp.reshape(
            ENT_RATIO * (sacc[0] * navg - avg_ent), (1, 1))


def _ent(dist):
    n_blocks = N_TOK // BLK
    kern = functools.partial(_ent_block, n_blocks=n_blocks)
    return pl.pallas_call(
        kern,
        grid=(n_blocks,),
        in_specs=[pl.BlockSpec((BLK, K), lambda i: (i, 0))],
        out_specs=[pl.BlockSpec((1, 1), lambda i: (0, 0))],
        out_shape=[jax.ShapeDtypeStruct((1, 1), jnp.float32)],
        scratch_shapes=[
            pltpu.VMEM((1, K), jnp.float32),
            pltpu.SMEM((2,), jnp.float32),
        ],
    )(dist)[0]


def _topk_body(dist_hbm, bm_hbm, ti_hbm, tw_hbm,
               dbuf, bbuf, blkids, cidx, cvals, tibuf, twbuf, sem):
    wid = lax.axis_index("s") * 2 + lax.axis_index("c")
    lane = lax.iota(jnp.int32, L)
    lane8 = (lane & 7) * NB
    inf_v = jnp.full((L,), jnp.inf, jnp.float32)
    n_chunks = TPW // CT

    def chunk_start(ci, buf):
        tok0 = wid * TPW + ci * CT
        pltpu.async_copy(dist_hbm.at[pl.ds(tok0, CT)], dbuf.at[buf], sem)
        pltpu.async_copy(bm_hbm.at[pl.ds(tok0, CT)], bbuf.at[buf], sem)

    chunk_start(0, 0)

    def chunk_body(ci, _):
        tok0 = wid * TPW + ci * CT
        b = ci % 2

        @pl.when(ci < n_chunks - 1)
        def _prefetch():
            chunk_start(ci + 1, (ci + 1) % 2)

        pltpu.make_async_copy(
            dist_hbm.at[pl.ds(tok0, CT)], dbuf.at[b], sem).wait()
        pltpu.make_async_copy(
            bm_hbm.at[pl.ds(tok0, CT)], bbuf.at[b], sem).wait()

        def token_body(t, _):
            t_v = jnp.broadcast_to(t, (L,))
            b_v = jnp.broadcast_to(b, (L,))

            bms = [bbuf[b, t, pl.ds(g * L, L)] for g in range(NB // L)]
            p0 = jnp.minimum(jnp.minimum(bms[0], bms[1]),
                             jnp.minimum(bms[2], bms[3]))
            p1 = jnp.minimum(jnp.minimum(bms[4], bms[5]),
                             jnp.minimum(bms[6], bms[7]))
            sk, _sv = plsc.sort_key_val(jnp.minimum(p0, p1), lane)
            thr_v = jnp.broadcast_to(sk[H - 1], (L,))

            nb = jnp.int32(0)
            for g in range(NB // L):
                mg = bms[g] <= thr_v
                plsc.store_compressed(blkids.at[pl.ds(nb, L)],
                                      lane + g * L, mask=mg)
                nb = nb + plsc.all_reduce_population_count(mg)[0]

            nb_v = jnp.broadcast_to(nb, (L,))

            def blk_body(i, cnt):
                i_v = jnp.broadcast_to(2 * i, (L,))
                blk0 = plsc.load_gather(blkids, [i_v])
                blk1 = plsc.load_gather(blkids, [i_v + 1])
                k0 = blk0 + lane8
                k1 = jnp.minimum(jnp.maximum(blk1, jnp.int32(0)),
                                 jnp.int32(NB - 1)) + lane8
                v0 = plsc.load_gather(dbuf, [b_v, t_v, k0])
                v1 = plsc.load_gather(dbuf, [b_v, t_v, k1])
                m0 = jnp.logical_and(v0 <= thr_v, lane < SE)
                m1 = jnp.logical_and(
                    jnp.logical_and(v1 <= thr_v, lane < SE),
                    i_v + 1 < nb_v)
                plsc.store_compressed(cidx.at[pl.ds(cnt, L)], k0, mask=m0)
                c1 = cnt + plsc.all_reduce_population_count(m0)[0]
                plsc.store_compressed(cidx.at[pl.ds(c1, L)], k1, mask=m1)
                return c1 + plsc.all_reduce_population_count(m1)[0]
            cnt = lax.fori_loop(0, (nb + 1) // 2, blk_body, jnp.int32(0))

            def fast_path(_):
                iv = cidx[pl.ds(0, L)]
                giv = jnp.where(lane < cnt, iv, 0)
                v = plsc.load_gather(dbuf, [b_v, t_v, giv])
                v = jnp.where(lane < cnt, v, jnp.inf)
                iv2 = jnp.where(lane < cnt, iv, jnp.int32(2048))
                sk2, sidx = plsc.sort_key_val(v, iv2)
                nxt = sk2.at[jnp.minimum(lane + 1, jnp.int32(L - 1))].get(
                    mode="promise_in_bounds")
                tiem = jnp.logical_and(sk2 == nxt, lane < H)
                anytie = plsc.all_reduce_population_count(tiem)[0]

                def notie(_a):
                    return sk2, sidx

                def tiecase(_a):
                    siv, _sl = plsc.sort_key_val(iv2, lane)
                    gv = jnp.minimum(siv, jnp.int32(K - 1))
                    vv = plsc.load_gather(dbuf, [b_v, t_v, gv])
                    vv = jnp.where(siv < 2048, vv, jnp.inf)
                    vs, _s2 = plsc.sort_key_val(vv, lane)
                    used = siv >= 2048
                    tidx = jnp.zeros((L,), jnp.int32)
                    for r in range(H):
                        srv = jnp.broadcast_to(vs[r], (L,))
                        hit = jnp.logical_and(vv == srv,
                                              jnp.logical_not(used))
                        p_v = plsc.all_reduce_ffs(hit)
                        used = jnp.logical_or(used, lane == p_v)
                        oi = siv.at[p_v].get(mode="promise_in_bounds")
                        tidx = jnp.where(lane == r, oi, tidx)
                    return vs, tidx

                return lax.cond(anytie == 0, notie, tiecase, 0)

            def gen_path(_):
                plsc.store_scatter(cidx, [lane + cnt],
                                   jnp.full((L,), K - 1, jnp.int32))
                nv = (cnt + 15) // 16

                def fill_body(j, _c):
                    civ = cidx[pl.ds(j * L, L)]
                    v = plsc.load_gather(dbuf, [b_v, t_v, civ])
                    off = jnp.where(lane + j * L < cnt, 0.0, jnp.inf)
                    cvals[pl.ds(j * L, L)] = v + off
                    return 0
                lax.fori_loop(0, nv, fill_body, 0)

                tvals = inf_v
                tidx = jnp.zeros((L,), jnp.int32)
                for r in range(H):
                    def min_body(j, mv):
                        return jnp.minimum(mv, cvals[pl.ds(j * L, L)])
                    mv = lax.fori_loop(0, nv, min_body, inf_v)
                    s_v = jnp.broadcast_to(jnp.min(mv), (L,))

                    def oi_body(j, pv):
                        hit = cvals[pl.ds(j * L, L)] == s_v
                        civ = cidx[pl.ds(j * L, L)]
                        return jnp.minimum(pv,
                                           jnp.where(hit, civ, jnp.int32(K)))
                    pv = lax.fori_loop(0, nv, oi_body,
                                       jnp.full((L,), K, jnp.int32))
                    oi_v = jnp.broadcast_to(jnp.min(pv), (L,))

                    def kill_body(j, _c):
                        hit2 = cidx[pl.ds(j * L, L)] == oi_v
                        plsc.store_scatter(cvals, [lane + j * L], inf_v,
                                           mask=hit2)
                        return 0
                    lax.fori_loop(0, nv, kill_body, 0)
                    tvals = jnp.where(lane == r, s_v, tvals)
                    tidx = jnp.where(lane == r, oi_v, tidx)
                return tvals, tidx

            tvals, tidx = lax.cond(cnt <= L, fast_path, gen_path, 0)

            d0 = jnp.broadcast_to(tvals[0], (L,))
            e = jnp.where(lane < H, jnp.exp(d0 - tvals), 0.0)
            tw = e / jnp.broadcast_to(jnp.sum(e), (L,))
            plsc.store_scatter(tibuf, [t_v, lane], tidx, mask=lane < H)
            plsc.store_scatter(twbuf, [t_v, lane], tw, mask=lane < H)
            return 0

        lax.fori_loop(0, CT, token_body, 0)
        pltpu.sync_copy(tibuf, ti_hbm.at[pl.ds(tok0, CT)])
        pltpu.sync_copy(twbuf, tw_hbm.at[pl.ds(tok0, CT)])
        return 0

    lax.fori_loop(0, n_chunks, chunk_body, 0)


def _topk_sc(dist, bmin):
    mesh = plsc.VectorSubcoreMesh(core_axis_name="c", subcore_axis_name="s")
    f = functools.partial(
        pl.kernel,
        mesh=mesh,
        compiler_params=pltpu.CompilerParams(needs_layout_passes=False),
        out_type=[
            jax.ShapeDtypeStruct((N_TOK, H), jnp.int32),
            jax.ShapeDtypeStruct((N_TOK, H), jnp.float32),
        ],
        scratch_types=[
            pltpu.VMEM((2, CT, K), jnp.float32),
            pltpu.VMEM((2, CT, NB), jnp.float32),
            pltpu.VMEM((NB + L,), jnp.int32),
            pltpu.VMEM((K + L,), jnp.int32),
            pltpu.VMEM((K + L,), jnp.float32),
            pltpu.VMEM((CT, H), jnp.int32),
            pltpu.VMEM((CT, H), jnp.float32),
            pltpu.SemaphoreType.DMA,
        ],
    )(_topk_body)
    return f(dist, bmin)


def _finish_block(x_ref, cb_ref, ti_ref, tw_ref, ent_ref,
                  enc_ref, q_ref, loss_ref, sacc, *, n_blocks):
    i = pl.program_id(0)

    @pl.when(i == 0)
    def _init():
        sacc[0] = 0.0

    x = x_ref[...]
    cb = cb_ref[...]
    ti = ti_ref[...]
    tw = tw_ref[...]
    iota_k = lax.broadcasted_iota(jnp.int32, (BLK, K), 1)
    enc = jnp.zeros((BLK, K), jnp.float32)
    for h in range(H):
        enc = jnp.where(iota_k == ti[:, h:h + 1], tw[:, h:h + 1], enc)
    enc_ref[...] = enc
    q = lax.dot_general(enc, cb, (((1,), (0,)), ((), ())),
                        preferred_element_type=jnp.float32)
    q_ref[...] = q
    r = q - x
    sacc[0] += jnp.sum(r * r)

    @pl.when(i == n_blocks - 1)
    def _fin():
        mse = sacc[0] * (1.0 / (N_TOK * D))
        loss_ref[...] = jnp.reshape(
            (1.0 + COMMIT) * mse + ent_ref[0, 0], (1, 1))


def _finish(x2d, cb, ti, tw, ent):
    n_blocks = N_TOK // BLK
    kern = functools.partial(_finish_block, n_blocks=n_blocks)
    return pl.pallas_call(
        kern,
        grid=(n_blocks,),
        in_specs=[
            pl.BlockSpec((BLK, D), lambda i: (i, 0)),
            pl.BlockSpec((K, D), lambda i: (0, 0)),
            pl.BlockSpec((BLK, H), lambda i: (i, 0)),
            pl.BlockSpec((BLK, H), lambda i: (i, 0)),
            pl.BlockSpec((1, 1), lambda i: (0, 0)),
        ],
        out_specs=[
            pl.BlockSpec((BLK, K), lambda i: (i, 0)),
            pl.BlockSpec((BLK, D), lambda i: (i, 0)),
            pl.BlockSpec((1, 1), lambda i: (0, 0)),
        ],
        out_shape=[
            jax.ShapeDtypeStruct((N_TOK, K), jnp.float32),
            jax.ShapeDtypeStruct((N_TOK, D), jnp.float32),
            jax.ShapeDtypeStruct((1, 1), jnp.float32),
        ],
        scratch_shapes=[
            pltpu.SMEM((2,), jnp.float32),
        ],
    )(x2d, cb, ti, tw, ent)


@jax.jit
def _vq(x2d, cb):
    dist, bmin = _dist(x2d, cb)
    ti, tw = _topk_sc(dist, bmin)
    ent = _ent(dist)
    enc, q, loss = _finish(x2d, cb, ti, tw, ent)
    return q, loss, ti, tw, enc


def kernel(x, codebook):
    b, t, d = x.shape
    x2d = x.reshape(b * t, d)
    q, loss, ti, tw, enc = _vq(x2d, codebook)
    return (q.reshape(b, t, d), loss[0, 0], ti.reshape(b, t, H),
            tw.reshape(b, t, H), enc.reshape(b, t, K))


# speedup vs baseline: 2.5708x; 1.1424x over previous
import functools

import jax
import jax.numpy as jnp
from jax import lax
from jax.experimental import pallas as pl
from jax.experimental.pallas import tpu as pltpu
from jax.experimental.pallas import tpu_sc as plsc

K = 1024
D = 256
H = 8
BLK = 256
N_TOK = 4096
TAU = 1.0
COMMIT = 0.25
ENT_RATIO = 0.1
ENT_TEMP = 0.01

NB = 128
SE = K // NB
NW = 32
TPW = N_TOK // NW
CT = 32
L = 16


def _dist_block(x_ref, cb_ref, d_ref, bm_ref):
    x = x_ref[...]
    cb = cb_ref[...]
    ab = lax.dot_general(x, cb, (((1,), (1,)), ((), ())),
                         preferred_element_type=jnp.float32)
    x2 = jnp.sum(x * x, axis=1, keepdims=True)
    b2 = jnp.sum(cb * cb, axis=1)[None, :]
    d = x2 - 2.0 * ab + b2
    d_ref[...] = d
    bm = d[:, 0:NB]
    for j in range(1, K // NB):
        bm = jnp.minimum(bm, d[:, NB * j:NB * (j + 1)])
    bm_ref[...] = bm


def _dist(x2d, cb):
    n_blocks = N_TOK // BLK
    return pl.pallas_call(
        _dist_block,
        grid=(n_blocks,),
        in_specs=[
            pl.BlockSpec((BLK, D), lambda i: (i, 0)),
            pl.BlockSpec((K, D), lambda i: (0, 0)),
        ],
        out_specs=[
            pl.BlockSpec((BLK, K), lambda i: (i, 0)),
            pl.BlockSpec((BLK, NB), lambda i: (i, 0)),
        ],
        out_shape=[
            jax.ShapeDtypeStruct((N_TOK, K), jnp.float32),
            jax.ShapeDtypeStruct((N_TOK, NB), jnp.float32),
        ],
    )(x2d, cb)


def _ent_block(d_ref, ent_ref, avgp_acc, sacc, *, n_blocks):
    i = pl.program_id(0)

    @pl.when(i == 0)
    def _init():
        avgp_acc[...] = jnp.zeros_like(avgp_acc)
        sacc[0] = 0.0

    d = d_ref[...]
    a = d * (-1.0 / ENT_TEMP)
    m = jnp.max(a, axis=1, keepdims=True)
    e = jnp.exp(a - m)
    z = jnp.sum(e, axis=1, keepdims=True)
    p = e / z
    s_ent = jnp.log(z[:, 0]) - jnp.sum(e * (a - m), axis=1) / z[:, 0]
    avgp_acc[...] += jnp.sum(p, axis=0, keepdims=True)
    sacc[0] += jnp.sum(s_ent)

    @pl.when(i == n_blocks - 1)
    def _fin():
        navg = 1.0 / N_TOK
        avg_p = avgp_acc[...] * navg
        avg_ent = -jnp.sum(avg_p * jnp.log(avg_p + 1e-5))
        ent_ref[...] = jnp.reshape(
            ENT_RATIO * (sacc[0] * navg - avg_ent), (1, 1))


def _ent(dist):
    n_blocks = N_TOK // BLK
    kern = functools.partial(_ent_block, n_blocks=n_blocks)
    return pl.pallas_call(
        kern,
        grid=(n_blocks,),
        in_specs=[pl.BlockSpec((BLK, K), lambda i: (i, 0))],
        out_specs=[pl.BlockSpec((1, 1), lambda i: (0, 0))],
        out_shape=[jax.ShapeDtypeStruct((1, 1), jnp.float32)],
        scratch_shapes=[
            pltpu.VMEM((1, K), jnp.float32),
            pltpu.SMEM((2,), jnp.float32),
        ],
    )(dist)[0]


def _topk_body(dist_hbm, bm_hbm, ti_hbm, tw_hbm,
               dbuf, bbuf, blkids, cidx, cvals, tibuf, twbuf, sem):
    wid = lax.axis_index("s") * 2 + lax.axis_index("c")
    lane = lax.iota(jnp.int32, L)
    lane8 = (lane & 7) * NB
    inf_v = jnp.full((L,), jnp.inf, jnp.float32)
    n_chunks = TPW // CT

    def chunk_start(ci, buf):
        tok0 = wid * TPW + ci * CT
        pltpu.async_copy(dist_hbm.at[pl.ds(tok0, CT)], dbuf.at[buf], sem)
        pltpu.async_copy(bm_hbm.at[pl.ds(tok0, CT)], bbuf.at[buf], sem)

    chunk_start(0, 0)

    def chunk_body(ci, _):
        tok0 = wid * TPW + ci * CT
        b = ci % 2

        @pl.when(ci < n_chunks - 1)
        def _prefetch():
            chunk_start(ci + 1, (ci + 1) % 2)

        pltpu.make_async_copy(
            dist_hbm.at[pl.ds(tok0, CT)], dbuf.at[b], sem).wait()
        pltpu.make_async_copy(
            bm_hbm.at[pl.ds(tok0, CT)], bbuf.at[b], sem).wait()

        def token_body(t, _):
            t_v = jnp.broadcast_to(t, (L,))
            b_v = jnp.broadcast_to(b, (L,))

            bms = [bbuf[b, t, pl.ds(g * L, L)] for g in range(NB // L)]
            p0 = jnp.minimum(jnp.minimum(bms[0], bms[1]),
                             jnp.minimum(bms[2], bms[3]))
            p1 = jnp.minimum(jnp.minimum(bms[4], bms[5]),
                             jnp.minimum(bms[6], bms[7]))
            sk, _sv = plsc.sort_key_val(jnp.minimum(p0, p1), lane)
            thr_v = jnp.broadcast_to(sk[H - 1], (L,))

            nb = jnp.int32(0)
            for g in range(NB // L):
                mg = bms[g] <= thr_v
                plsc.store_compressed(blkids.at[pl.ds(nb, L)],
                                      lane + g * L, mask=mg)
                nb = nb + plsc.all_reduce_population_count(mg)[0]

            nb_v = jnp.broadcast_to(nb, (L,))

            def blk_body(i, cnt):
                i2 = jnp.broadcast_to(2 * i, (L,))
                sel = jnp.where(lane < SE, i2, i2 + 1)
                blk = plsc.load_gather(blkids, [sel])
                blk = jnp.minimum(jnp.maximum(blk, jnp.int32(0)),
                                  jnp.int32(NB - 1))
                kpos = blk + lane8
                v = plsc.load_gather(dbuf, [b_v, t_v, kpos])
                msk = jnp.logical_and(v <= thr_v, sel < nb_v)
                plsc.store_compressed(cidx.at[pl.ds(cnt, L)], kpos, mask=msk)
                return cnt + plsc.all_reduce_population_count(msk)[0]
            cnt = lax.fori_loop(0, (nb + 1) // 2, blk_body, jnp.int32(0))

            def fast_path(_):
                iv = cidx[pl.ds(0, L)]
                giv = jnp.where(lane < cnt, iv, 0)
                v = plsc.load_gather(dbuf, [b_v, t_v, giv])
                v = jnp.where(lane < cnt, v, jnp.inf)
                iv2 = jnp.where(lane < cnt, iv, jnp.int32(2048))
                sk2, sidx = plsc.sort_key_val(v, iv2)
                nxt = sk2.at[jnp.minimum(lane + 1, jnp.int32(L - 1))].get(
                    mode="promise_in_bounds")
                tiem = jnp.logical_and(sk2 == nxt, lane < H)
                anytie = plsc.all_reduce_population_count(tiem)[0]

                def notie(_a):
                    return sk2, sidx

                def tiecase(_a):
                    siv, _sl = plsc.sort_key_val(iv2, lane)
                    gv = jnp.minimum(siv, jnp.int32(K - 1))
                    vv = plsc.load_gather(dbuf, [b_v, t_v, gv])
                    vv = jnp.where(siv < 2048, vv, jnp.inf)
                    vs, _s2 = plsc.sort_key_val(vv, lane)
                    used = siv >= 2048
                    tidx = jnp.zeros((L,), jnp.int32)
                    for r in range(H):
                        srv = jnp.broadcast_to(vs[r], (L,))
                        hit = jnp.logical_and(vv == srv,
                                              jnp.logical_not(used))
                        p_v = plsc.all_reduce_ffs(hit)
                        used = jnp.logical_or(used, lane == p_v)
                        oi = siv.at[p_v].get(mode="promise_in_bounds")
                        tidx = jnp.where(lane == r, oi, tidx)
                    return vs, tidx

                return lax.cond(anytie == 0, notie, tiecase, 0)

            def gen_path(_):
                plsc.store_scatter(cidx, [lane + cnt],
                                   jnp.full((L,), K - 1, jnp.int32))
                nv = (cnt + 15) // 16

                def fill_body(j, _c):
                    civ = cidx[pl.ds(j * L, L)]
                    v = plsc.load_gather(dbuf, [b_v, t_v, civ])
                    off = jnp.where(lane + j * L < cnt, 0.0, jnp.inf)
                    cvals[pl.ds(j * L, L)] = v + off
                    return 0
                lax.fori_loop(0, nv, fill_body, 0)

                tvals = inf_v
                tidx = jnp.zeros((L,), jnp.int32)
                for r in range(H):
                    def min_body(j, mv):
                        return jnp.minimum(mv, cvals[pl.ds(j * L, L)])
                    mv = lax.fori_loop(0, nv, min_body, inf_v)
                    s_v = jnp.broadcast_to(jnp.min(mv), (L,))

                    def oi_body(j, pv):
                        hit = cvals[pl.ds(j * L, L)] == s_v
                        civ = cidx[pl.ds(j * L, L)]
                        return jnp.minimum(pv,
                                           jnp.where(hit, civ, jnp.int32(K)))
                    pv = lax.fori_loop(0, nv, oi_body,
                                       jnp.full((L,), K, jnp.int32))
                    oi_v = jnp.broadcast_to(jnp.min(pv), (L,))

                    def kill_body(j, _c):
                        hit2 = cidx[pl.ds(j * L, L)] == oi_v
                        plsc.store_scatter(cvals, [lane + j * L], inf_v,
                                           mask=hit2)
                        return 0
                    lax.fori_loop(0, nv, kill_body, 0)
                    tvals = jnp.where(lane == r, s_v, tvals)
                    tidx = jnp.where(lane == r, oi_v, tidx)
                return tvals, tidx

            tvals, tidx = lax.cond(cnt <= L, fast_path, gen_path, 0)

            d0 = jnp.broadcast_to(tvals[0], (L,))
            e = jnp.where(lane < H, jnp.exp(d0 - tvals), 0.0)
            tw = e / jnp.broadcast_to(jnp.sum(e), (L,))
            plsc.store_scatter(tibuf, [t_v, lane], tidx, mask=lane < H)
            plsc.store_scatter(twbuf, [t_v, lane], tw, mask=lane < H)
            return 0

        lax.fori_loop(0, CT, token_body, 0)
        pltpu.sync_copy(tibuf, ti_hbm.at[pl.ds(tok0, CT)])
        pltpu.sync_copy(twbuf, tw_hbm.at[pl.ds(tok0, CT)])
        return 0

    lax.fori_loop(0, n_chunks, chunk_body, 0)


def _topk_sc(dist, bmin):
    mesh = plsc.VectorSubcoreMesh(core_axis_name="c", subcore_axis_name="s")
    f = functools.partial(
        pl.kernel,
        mesh=mesh,
        compiler_params=pltpu.CompilerParams(needs_layout_passes=False),
        out_type=[
            jax.ShapeDtypeStruct((N_TOK, H), jnp.int32),
            jax.ShapeDtypeStruct((N_TOK, H), jnp.float32),
        ],
        scratch_types=[
            pltpu.VMEM((2, CT, K), jnp.float32),
            pltpu.VMEM((2, CT, NB), jnp.float32),
            pltpu.VMEM((NB + L,), jnp.int32),
            pltpu.VMEM((K + L,), jnp.int32),
            pltpu.VMEM((K + L,), jnp.float32),
            pltpu.VMEM((CT, H), jnp.int32),
            pltpu.VMEM((CT, H), jnp.float32),
            pltpu.SemaphoreType.DMA,
        ],
    )(_topk_body)
    return f(dist, bmin)


def _finish_block(x_ref, cb_ref, ti_ref, tw_ref, ent_ref,
                  enc_ref, q_ref, loss_ref, sacc, *, n_blocks):
    i = pl.program_id(0)

    @pl.when(i == 0)
    def _init():
        sacc[0] = 0.0

    x = x_ref[...]
    cb = cb_ref[...]
    ti = ti_ref[...]
    tw = tw_ref[...]
    iota_k = lax.broadcasted_iota(jnp.int32, (BLK, K), 1)
    enc = jnp.zeros((BLK, K), jnp.float32)
    for h in range(H):
        enc = jnp.where(iota_k == ti[:, h:h + 1], tw[:, h:h + 1], enc)
    enc_ref[...] = enc
    q = lax.dot_general(enc, cb, (((1,), (0,)), ((), ())),
                        preferred_element_type=jnp.float32)
    q_ref[...] = q
    r = q - x
    sacc[0] += jnp.sum(r * r)

    @pl.when(i == n_blocks - 1)
    def _fin():
        mse = sacc[0] * (1.0 / (N_TOK * D))
        loss_ref[...] = jnp.reshape(
            (1.0 + COMMIT) * mse + ent_ref[0, 0], (1, 1))


def _finish(x2d, cb, ti, tw, ent):
    n_blocks = N_TOK // BLK
    kern = functools.partial(_finish_block, n_blocks=n_blocks)
    return pl.pallas_call(
        kern,
        grid=(n_blocks,),
        in_specs=[
            pl.BlockSpec((BLK, D), lambda i: (i, 0)),
            pl.BlockSpec((K, D), lambda i: (0, 0)),
            pl.BlockSpec((BLK, H), lambda i: (i, 0)),
            pl.BlockSpec((BLK, H), lambda i: (i, 0)),
            pl.BlockSpec((1, 1), lambda i: (0, 0)),
        ],
        out_specs=[
            pl.BlockSpec((BLK, K), lambda i: (i, 0)),
            pl.BlockSpec((BLK, D), lambda i: (i, 0)),
            pl.BlockSpec((1, 1), lambda i: (0, 0)),
        ],
        out_shape=[
            jax.ShapeDtypeStruct((N_TOK, K), jnp.float32),
            jax.ShapeDtypeStruct((N_TOK, D), jnp.float32),
            jax.ShapeDtypeStruct((1, 1), jnp.float32),
        ],
        scratch_shapes=[
            pltpu.SMEM((2,), jnp.float32),
        ],
    )(x2d, cb, ti, tw, ent)


@jax.jit
def _vq(x2d, cb):
    dist, bmin = _dist(x2d, cb)
    ti, tw = _topk_sc(dist, bmin)
    ent = _ent(dist)
    enc, q, loss = _finish(x2d, cb, ti, tw, ent)
    return q, loss, ti, tw, enc


def kernel(x, codebook):
    b, t, d = x.shape
    x2d = x.reshape(b * t, d)
    q, loss, ti, tw, enc = _vq(x2d, codebook)
    return (q.reshape(b, t, d), loss[0, 0], ti.reshape(b, t, H),
            tw.reshape(b, t, H), enc.reshape(b, t, K))
